# baseline (speedup 1.0000x reference)
import functools

import jax
import jax.numpy as jnp
from jax import lax
from jax.experimental import pallas as pl
from jax.experimental.pallas import tpu as pltpu
from jax.experimental.pallas import tpu_sc as plsc

_NC = 2
_NS = 16
_NW = _NC * _NS
_CT = 8
_NBUF = 2


def kernel(x, pos_table):
    T, B, D = x.shape
    t_per_w = T // _NW
    n_chunks = t_per_w // _CT
    mesh = plsc.VectorSubcoreMesh(core_axis_name="c", subcore_axis_name="s")

    @functools.partial(
        pl.kernel,
        mesh=mesh,
        out_type=jax.ShapeDtypeStruct((T, B, D), jnp.float32),
        scratch_types=(
            [pltpu.VMEM((_CT, B, D), jnp.float32)] * _NBUF
            + [pltpu.VMEM((_CT, D), jnp.float32)] * _NBUF
            + [pltpu.VMEM_SHARED((_NS, _CT, B, D), jnp.float32)]
            + [pltpu.SemaphoreType.DMA] * (2 * _NBUF + 2)
        ),
    )
    def sc_add(x_hbm, pos_hbm, out_hbm, *scratch):
        xvs = scratch[:_NBUF]
        pvs = scratch[_NBUF:2 * _NBUF]
        spm = scratch[2 * _NBUF]
        sis = scratch[2 * _NBUF + 1:3 * _NBUF + 1]
        sos = scratch[3 * _NBUF + 1:4 * _NBUF + 1]
        sx = scratch[4 * _NBUF + 1]
        sh = scratch[4 * _NBUF + 2]
        sid = lax.axis_index("s")
        wid = sid * _NC + lax.axis_index("c")
        base = wid * t_per_w
        h = _CT // 2

        def start_in(ci, b):
            t0 = base + ci * _CT
            pltpu.async_copy(x_hbm.at[pl.ds(t0, h)], xvs[b].at[pl.ds(0, h)], sis[b])
            pltpu.async_copy(x_hbm.at[pl.ds(t0 + h, h)], xvs[b].at[pl.ds(h, h)], sis[b])
            pltpu.async_copy(pos_hbm.at[pl.ds(t0, _CT)], pvs[b], sis[b])

        def wait_in(b):
            pltpu.make_async_copy(x_hbm.at[pl.ds(base, _CT)], xvs[b], sis[b]).wait()
            pltpu.make_async_copy(pos_hbm.at[pl.ds(base, _CT)], pvs[b], sis[b]).wait()

        def start_out(ci, b):
            t0 = base + ci * _CT
            if b == 0:
                pltpu.async_copy(xvs[b], out_hbm.at[pl.ds(t0, _CT)], sos[b])
            else:
                @pl.when(ci >= 2 * _NBUF - 1)
                def _():
                    pltpu.make_async_copy(
                        spm.at[sid], out_hbm.at[pl.ds(base, _CT)], sh).wait()

                pltpu.async_copy(xvs[b], spm.at[sid], sx)

        def release_out(ci, nb):
            if nb == 0:
                pltpu.make_async_copy(
                    xvs[nb], out_hbm.at[pl.ds(base, _CT)], sos[nb]).wait()
            else:
                pltpu.make_async_copy(xvs[nb], spm.at[sid], sx).wait()
                t0 = base + (ci - 1) * _CT
                pltpu.async_copy(spm.at[sid], out_hbm.at[pl.ds(t0, _CT)], sh)

        def compute(b):
            xvb, pvb = xvs[b], pvs[b]

            @plsc.parallel_loop(0, _CT, 1)
            def _row(j):
                @plsc.parallel_loop(0, D, 16, unroll=8)
                def _lane(k0):
                    sl = pl.ds(k0, 16)
                    p = pvb[j, sl]
                    for bb in range(B):
                        plsc.addupdate(xvb.at[j, bb, sl], p)

        def emit_chunk(ci, b):
            nb = (b + 1) % _NBUF
            wait_in(b)

            @pl.when(ci >= _NBUF - 1)
            def _():
                release_out(ci, nb)

            @pl.when(ci + 1 < n_chunks)
            def _():
                start_in(ci + 1, nb)

            compute(b)
            start_out(ci, b)

        start_in(0, 0)

        def ring(g, carry):
            for b in range(_NBUF):
                emit_chunk(g * _NBUF + b, b)
            return carry

        lax.fori_loop(0, n_chunks // _NBUF, ring, 0)
        last_bounce = n_chunks - 1
        pltpu.make_async_copy(xvs[_NBUF - 1], spm.at[sid], sx).wait()
        t0 = base + last_bounce * _CT
        pltpu.async_copy(spm.at[sid], out_hbm.at[pl.ds(t0, _CT)], sh)
        pltpu.make_async_copy(spm.at[sid], out_hbm.at[pl.ds(base, _CT)], sh).wait()

    return sc_add(x, pos_table)


# speedup vs baseline: 1.0166x; 1.0166x over previous
import functools

import jax
import jax.numpy as jnp
from jax import lax
from jax.experimental import pallas as pl
from jax.experimental.pallas import tpu as pltpu
from jax.experimental.pallas import tpu_sc as plsc

_NC = 2
_NS = 16
_NW = _NC * _NS
_CT = 8
_NBUF = 3


def kernel(x, pos_table):
    T, B, D = x.shape
    t_per_w = T // _NW
    n_chunks = t_per_w // _CT
    mesh = plsc.VectorSubcoreMesh(core_axis_name="c", subcore_axis_name="s")

    @functools.partial(
        pl.kernel,
        mesh=mesh,
        out_type=jax.ShapeDtypeStruct((T, B, D), jnp.float32),
        scratch_types=(
            [pltpu.VMEM((_CT, B, D), jnp.float32)] * _NBUF
            + [pltpu.VMEM((_CT, D), jnp.float32)] * _NBUF
            + [pltpu.SemaphoreType.DMA] * (2 * _NBUF)
        ),
    )
    def sc_add(x_hbm, pos_hbm, out_hbm, *scratch):
        xvs = scratch[:_NBUF]
        pvs = scratch[_NBUF:2 * _NBUF]
        sis = scratch[2 * _NBUF:3 * _NBUF]
        sos = scratch[3 * _NBUF:4 * _NBUF]
        wid = lax.axis_index("s") * _NC + lax.axis_index("c")
        base = wid * t_per_w

        def start_in(ci, b):
            t0 = base + ci * _CT
            pltpu.async_copy(x_hbm.at[pl.ds(t0, _CT)], xvs[b], sis[b])
            pltpu.async_copy(pos_hbm.at[pl.ds(t0, _CT)], pvs[b], sis[b])

        def wait_in(b):
            pltpu.make_async_copy(x_hbm.at[pl.ds(base, _CT)], xvs[b], sis[b]).wait()
            pltpu.make_async_copy(pos_hbm.at[pl.ds(base, _CT)], pvs[b], sis[b]).wait()

        def start_out(ci, b):
            t0 = base + ci * _CT
            pltpu.async_copy(xvs[b], out_hbm.at[pl.ds(t0, _CT)], sos[b])

        def wait_out(b):
            pltpu.make_async_copy(xvs[b], out_hbm.at[pl.ds(base, _CT)], sos[b]).wait()

        def compute(b):
            xvb, pvb = xvs[b], pvs[b]

            @plsc.parallel_loop(0, _CT, 1)
            def _row(j):
                @plsc.parallel_loop(0, D, 16, unroll=16)
                def _lane(k0):
                    sl = pl.ds(k0, 16)
                    p = pvb[j, sl]
                    for bb in range(B):
                        plsc.addupdate(xvb.at[j, bb, sl], p)

        def emit_chunk(ci, b, last):
            nb = (b + 1) % _NBUF
            wait_in(b)

            @pl.when(ci >= _NBUF - 1)
            def _():
                wait_out(nb)

            if not last:
                @pl.when(ci + 1 < n_chunks)
                def _():
                    start_in(ci + 1, nb)

            compute(b)
            start_out(ci, b)

        start_in(0, 0)
        n_full = (n_chunks // _NBUF) * _NBUF

        def ring(g, carry):
            for b in range(_NBUF):
                emit_chunk(g * _NBUF + b, b, last=False)
            return carry

        lax.fori_loop(0, n_chunks // _NBUF, ring, 0)
        for ci in range(n_full, n_chunks):
            emit_chunk(ci, ci % _NBUF, last=(ci == n_chunks - 1))
        for ci in range(n_chunks - _NBUF + 1, n_chunks):
            wait_out(ci % _NBUF)

    return sc_add(x, pos_table)


# speedup vs baseline: 1.0350x; 1.0181x over previous
import functools

import jax
import jax.numpy as jnp
from jax import lax
from jax.experimental import pallas as pl
from jax.experimental.pallas import tpu as pltpu
from jax.experimental.pallas import tpu_sc as plsc

_NC = 2
_NS = 16
_NW = _NC * _NS
_CT = 8
_NBUF = 2


def kernel(x, pos_table):
    T, B, D = x.shape
    t_per_w = T // _NW
    n_chunks = t_per_w // _CT
    mesh = plsc.VectorSubcoreMesh(core_axis_name="c", subcore_axis_name="s")

    @functools.partial(
        pl.kernel,
        mesh=mesh,
        out_type=jax.ShapeDtypeStruct((T, B, D), jnp.float32),
        scratch_types=(
            [pltpu.VMEM((_CT, B, D), jnp.float32)] * _NBUF
            + [pltpu.VMEM((_CT, D), jnp.float32)] * _NBUF
            + [pltpu.SemaphoreType.DMA] * (2 * _NBUF)
        ),
    )
    def sc_add(x_hbm, pos_hbm, out_hbm, *scratch):
        xvs = scratch[:_NBUF]
        pvs = scratch[_NBUF:2 * _NBUF]
        sis = scratch[2 * _NBUF:3 * _NBUF]
        sos = scratch[3 * _NBUF:4 * _NBUF]
        wid = lax.axis_index("s") * _NC + lax.axis_index("c")
        base = wid * t_per_w

        def start_in(ci, b):
            t0 = base + ci * _CT
            pltpu.async_copy(x_hbm.at[pl.ds(t0, _CT)], xvs[b], sis[b])
            pltpu.async_copy(pos_hbm.at[pl.ds(t0, _CT)], pvs[b], sis[b])

        def wait_in(b):
            pltpu.make_async_copy(x_hbm.at[pl.ds(base, _CT)], xvs[b], sis[b]).wait()
            pltpu.make_async_copy(pos_hbm.at[pl.ds(base, _CT)], pvs[b], sis[b]).wait()

        def start_out(ci, b):
            t0 = base + ci * _CT
            pltpu.async_copy(xvs[b], out_hbm.at[pl.ds(t0, _CT)], sos[b])

        def wait_out(b):
            pltpu.make_async_copy(xvs[b], out_hbm.at[pl.ds(base, _CT)], sos[b]).wait()

        def compute(b):
            xvb, pvb = xvs[b], pvs[b]

            @plsc.parallel_loop(0, _CT, 1)
            def _row(j):
                @plsc.parallel_loop(0, D, 16, unroll=8)
                def _lane(k0):
                    sl = pl.ds(k0, 16)
                    p = pvb[j, sl]
                    for bb in range(B):
                        plsc.addupdate(xvb.at[j, bb, sl], p)

        def emit_chunk(ci, b, last):
            nb = (b + 1) % _NBUF
            wait_in(b)

            @pl.when(ci >= _NBUF - 1)
            def _():
                wait_out(nb)

            if not last:
                @pl.when(ci + 1 < n_chunks)
                def _():
                    start_in(ci + 1, nb)

            compute(b)
            start_out(ci, b)

        start_in(0, 0)
        n_full = (n_chunks // _NBUF) * _NBUF

        def ring(g, carry):
            for b in range(_NBUF):
                emit_chunk(g * _NBUF + b, b, last=False)
            return carry

        lax.fori_loop(0, n_chunks // _NBUF, ring, 0)
        for ci in range(n_full, n_chunks):
            emit_chunk(ci, ci % _NBUF, last=(ci == n_chunks - 1))
        for ci in range(n_chunks - _NBUF + 1, n_chunks):
            wait_out(ci % _NBUF)

    return sc_add(x, pos_table)
